# baseline (speedup 1.0000x reference)
import functools

import jax
import jax.numpy as jnp
from jax import lax
from jax.experimental import pallas as pl
from jax.experimental.pallas import tpu as pltpu
from jax.experimental.pallas import tpu_sc as plsc

N = 10000
CH = 128
E = 320000
NC, NS, L = 2, 16, 16
EPT = E // NS
CHUNK = 80
SUP = 2000
NCPS = SUP // CHUNK
NSUP = EPT // SUP
NTRI = 6
RPT = 640


def _sc_body(xi, hi, rowi, coli, wi, agg_out, dis_out,
             acc, deg, dis_v, row_v, w_v, col2_v,
             rows0, rows1, rows2, gs0, gs1, gs2, ss0, ss1, ss2, asem):
    cid = lax.axis_index("c")
    sid = lax.axis_index("s")
    zero16 = jnp.zeros((L,), jnp.float32)
    rows = (rows0, rows1, rows2)
    gsems = (gs0, gs1, gs2)
    ssems = (ss0, ss1, ss2)

    def _zr(e, carry):
        for j in range(CH // L):
            rows0[e, pl.ds(j * L, L)] = zero16
        return carry
    lax.fori_loop(0, CHUNK, _zr, 0)

    @pl.when(sid < NS - 1)
    def _():
        for j in range(RPT // CHUNK):
            pltpu.sync_copy(rows0, acc.at[pl.ds(sid * RPT + j * CHUNK, CHUNK)])
            pltpu.sync_copy(rows0.at[0, pl.ds(0, CHUNK)],
                            deg.at[pl.ds(sid * RPT + j * CHUNK, CHUNK)])

    @pl.when(sid == NS - 1)
    def _():
        for j in range((N - (NS - 1) * RPT) // CHUNK):
            pltpu.sync_copy(rows0, acc.at[pl.ds((NS - 1) * RPT + j * CHUNK, CHUNK)])
            pltpu.sync_copy(rows0.at[0, pl.ds(0, CHUNK)],
                            deg.at[pl.ds((NS - 1) * RPT + j * CHUNK, CHUNK)])

    ebase = sid * EPT

    def _stage_col2(s, carry):
        pltpu.sync_copy(coli.at[pl.ds(ebase + s * SUP, SUP)], row_v)

        def _bld(j, carry2):
            for k in range(CHUNK // L):
                col2_v[j, pl.ds(k * L, L)] = row_v[pl.ds(j * CHUNK + k * L, L)]
            return carry2
        lax.fori_loop(0, NCPS, _bld, carry)
        return carry

    plsc.subcore_barrier()

    def _deg_sup(s, carry):
        _stage_col2(s, 0)
        pltpu.sync_copy(wi.at[pl.ds(ebase + s * SUP, SUP)], w_v)

        def _fire(c, carry2):
            pltpu.async_copy(w_v.at[pl.ds(c * CHUNK, CHUNK)],
                             deg.at[col2_v.at[c]], asem, add=True)
            return carry2
        lax.fori_loop(0, NCPS, _fire, 0)

        def _drain(c, carry2):
            pltpu.make_async_copy(w_v.at[pl.ds(c * CHUNK, CHUNK)],
                                  deg.at[col2_v.at[c]], asem).wait()
            return carry2
        lax.fori_loop(0, NCPS, _drain, 0)
        return carry
    lax.fori_loop(0, NSUP, _deg_sup, 0)

    plsc.subcore_barrier()

    pltpu.sync_copy(deg, dis_v)

    def _dis(g, carry):
        d = dis_v[pl.ds(g * L, L)] + 1.0
        i = lax.bitcast_convert_type(d, jnp.int32)
        i = 0x5F3759DF - lax.shift_right_arithmetic(i, 1)
        y = lax.bitcast_convert_type(i, jnp.float32)
        y = y * (1.5 - 0.5 * d * y * y)
        y = y * (1.5 - 0.5 * d * y * y)
        y = y * (1.5 - 0.5 * d * y * y)
        dis_v[pl.ds(g * L, L)] = y
        return carry
    lax.fori_loop(0, N // L, _dis, 0)

    def _gissue(tab, c, p):
        pltpu.async_copy(tab.at[row_v.at[pl.ds(c * CHUNK, CHUNK)]],
                         rows[p], gsems[p])

    def _gwait(tab, c, p):
        pltpu.make_async_copy(tab.at[row_v.at[pl.ds(c * CHUNK, CHUNK)]],
                              rows[p], gsems[p]).wait()

    def _sissue(c, p):
        pltpu.async_copy(rows[p], acc.at[col2_v.at[c]], ssems[p], add=True)

    def _swait(c, p):
        pltpu.make_async_copy(rows[p], acc.at[col2_v.at[c]], ssems[p]).wait()

    def _scale(c, p):
        def _sc1(q, carry):
            e0 = q * 4
            for de in range(4):
                e = e0 + de
                nb = plsc.load_gather(
                    w_v, [jnp.full((L,), c * CHUNK + e, jnp.int32)])
                for j in range(CH // L):
                    rows[p][e, pl.ds(j * L, L)] = rows[p][e, pl.ds(j * L, L)] * nb
            return carry
        lax.fori_loop(0, CHUNK // 4, _sc1, 0)

    def _make_sup(tab):
      def _sup(s, carry):
        _stage_col2(s, 0)
        pltpu.sync_copy(wi.at[pl.ds(ebase + s * SUP, SUP)], w_v)

        def _ncol(j, carry2):
            for k in range(CHUNK // L):
                d0 = pl.ds(j * CHUNK + k * L, L)
                dc = plsc.load_gather(dis_v, [row_v[d0]])
                w_v[d0] = w_v[d0] * dc
            return carry2
        lax.fori_loop(0, NCPS, _ncol, 0)

        pltpu.sync_copy(rowi.at[pl.ds(ebase + s * SUP, SUP)], row_v)

        def _nrow(j, carry2):
            for k in range(CHUNK // L):
                d0 = pl.ds(j * CHUNK + k * L, L)
                dr = plsc.load_gather(dis_v, [row_v[d0]])
                w_v[d0] = w_v[d0] * dr
            return carry2
        lax.fori_loop(0, NCPS, _nrow, 0)

        _gissue(tab, 0, 0)
        _gissue(tab, 1, 1)
        _gissue(tab, 2, 2)
        _gwait(tab, 0, 0); _scale(0, 0); _sissue(0, 0)
        _swait(0, 0); _gissue(tab, 3, 0)
        _gwait(tab, 1, 1); _scale(1, 1); _sissue(1, 1)
        _swait(1, 1); _gissue(tab, 4, 1)
        _gwait(tab, 2, 2); _scale(2, 2); _sissue(2, 2)

        def _tri(t, carry2):
            for k in range(3):
                c = 3 * t + k
                _swait(c - 1, (k + 2) % 3)
                _gissue(tab, c + 2, (k + 2) % 3)
                _gwait(tab, c, k)
                _scale(c, k)
                _sissue(c, k)
            return carry2
        lax.fori_loop(1, NTRI + 1, _tri, 0)

        for c in range(3 * NTRI + 3, NCPS):
            p = c % 3
            _swait(c - 1, (c + 2) % 3)
            if c + 2 < NCPS:
                _gissue(tab, c + 2, (c + 2) % 3)
            _gwait(tab, c, p)
            _scale(c, p)
            _sissue(c, p)
        _swait(NCPS - 1, (NCPS - 1) % 3)
        return carry
      return _sup

    @pl.when(cid == 0)
    def _():
        lax.fori_loop(0, NSUP, _make_sup(xi), 0)

    @pl.when(cid == 1)
    def _():
        lax.fori_loop(0, NSUP, _make_sup(hi), 0)

    plsc.subcore_barrier()

    tail = N - (NS - 1) * RPT

    @pl.when(sid < NS - 1)
    def _():
        pltpu.sync_copy(acc.at[pl.ds(sid * RPT, RPT)],
                        agg_out.at[pl.ds(cid * N + sid * RPT, RPT)])

    @pl.when(sid == NS - 1)
    def _():
        pltpu.sync_copy(acc.at[pl.ds((NS - 1) * RPT, tail)],
                        agg_out.at[pl.ds(cid * N + (NS - 1) * RPT, tail)])

    @pl.when(cid == 0)
    def _():
        @pl.when(sid < NS - 1)
        def _():
            pltpu.sync_copy(dis_v.at[pl.ds(sid * RPT, RPT)],
                            dis_out.at[pl.ds(sid * RPT, RPT)])

        @pl.when(sid == NS - 1)
        def _():
            pltpu.sync_copy(dis_v.at[pl.ds((NS - 1) * RPT, tail)],
                            dis_out.at[pl.ds((NS - 1) * RPT, tail)])


_sc_agg = functools.partial(
    pl.kernel,
    out_type=(
        jax.ShapeDtypeStruct((2 * N, CH), jnp.float32),
        jax.ShapeDtypeStruct((N,), jnp.float32),
    ),
    mesh=plsc.VectorSubcoreMesh(core_axis_name="c", subcore_axis_name="s"),
    compiler_params=pltpu.CompilerParams(needs_layout_passes=False),
    scratch_types=[
        pltpu.VMEM_SHARED((N, CH), jnp.float32),
        pltpu.VMEM_SHARED((N,), jnp.float32),
        pltpu.VMEM((N,), jnp.float32),
        pltpu.VMEM((SUP,), jnp.int32),
        pltpu.VMEM((SUP,), jnp.float32),
        pltpu.VMEM((NCPS, CHUNK), jnp.int32),
        pltpu.VMEM((CHUNK, CH), jnp.float32),
        pltpu.VMEM((CHUNK, CH), jnp.float32),
        pltpu.VMEM((CHUNK, CH), jnp.float32),
        pltpu.SemaphoreType.DMA,
        pltpu.SemaphoreType.DMA,
        pltpu.SemaphoreType.DMA,
        pltpu.SemaphoreType.DMA,
        pltpu.SemaphoreType.DMA,
        pltpu.SemaphoreType.DMA,
        pltpu.SemaphoreType.DMA,
    ],
)(_sc_body)


BN = 2000


def _tc_body(aggx, aggh, x, h, d, ct, wx, wh, bb, hn, cn):
    d2 = d[...] * d[...]
    ax = aggx[...] + d2 * x[...]
    ah = aggh[...] + d2 * h[...]
    cc = jnp.dot(ax, wx[...], preferred_element_type=jnp.float32)
    cc = cc + jnp.dot(ah, wh[...], preferred_element_type=jnp.float32)
    cc = cc + bb[...]
    ig = jax.nn.sigmoid(cc[:, :CH])
    fg = jax.nn.sigmoid(cc[:, CH:2 * CH])
    og = jax.nn.sigmoid(cc[:, 2 * CH:3 * CH])
    gg = jnp.tanh(cc[:, 3 * CH:])
    c2 = fg * ct[...] + ig * gg
    hn[...] = og * jnp.tanh(c2)
    cn[...] = c2


_tc_cell = pl.pallas_call(
    _tc_body,
    grid=(N // BN,),
    in_specs=[
        pl.BlockSpec((BN, CH), lambda i: (i, 0)),
        pl.BlockSpec((BN, CH), lambda i: (N // BN + i, 0)),
        pl.BlockSpec((BN, CH), lambda i: (i, 0)),
        pl.BlockSpec((BN, CH), lambda i: (i, 0)),
        pl.BlockSpec((BN, 1), lambda i: (i, 0)),
        pl.BlockSpec((BN, CH), lambda i: (i, 0)),
        pl.BlockSpec((CH, 4 * CH), lambda i: (0, 0)),
        pl.BlockSpec((CH, 4 * CH), lambda i: (0, 0)),
        pl.BlockSpec((1, 4 * CH), lambda i: (0, 0)),
    ],
    out_specs=[
        pl.BlockSpec((BN, CH), lambda i: (i, 0)),
        pl.BlockSpec((BN, CH), lambda i: (i, 0)),
    ],
    out_shape=[
        jax.ShapeDtypeStruct((N, CH), jnp.float32),
        jax.ShapeDtypeStruct((N, CH), jnp.float32),
    ],
)


def kernel(x, edge_index, edge_attr, h_t, c_t, W, b):
    row = edge_index[0]
    col = edge_index[1]
    agg, dis = _sc_agg(x, h_t, row, col, edge_attr)
    wt = W.T
    hn, cn = _tc_cell(agg, agg, x, h_t, dis[:, None], c_t,
                      wt[:CH], wt[CH:], b[None, :])
    return hn, cn


# speedup vs baseline: 1.1412x; 1.1412x over previous
import functools

import jax
import jax.numpy as jnp
from jax import lax
from jax.experimental import pallas as pl
from jax.experimental.pallas import tpu as pltpu
from jax.experimental.pallas import tpu_sc as plsc

N = 10000
CH = 128
E = 320000
NC, NS, L = 2, 16, 16
EPT = E // NS
CHUNK = 80
SUP = 2000
NCPS = SUP // CHUNK
NSUP = EPT // SUP
NTRI = 6
RPT = 640


def _sc_body(xi, hi, rowi, coli, wi, agg_out, dis_out,
             acc, deg, dis_v, row_v, w_v, col2_v,
             rows0, rows1, rows2, gs0, gs1, gs2, ss0, ss1, ss2, asem):
    cid = lax.axis_index("c")
    sid = lax.axis_index("s")
    zero16 = jnp.zeros((L,), jnp.float32)
    rows = (rows0, rows1, rows2)
    gsems = (gs0, gs1, gs2)
    ssems = (ss0, ss1, ss2)

    def _zr(e, carry):
        for j in range(CH // L):
            rows0[e, pl.ds(j * L, L)] = zero16
        return carry
    lax.fori_loop(0, CHUNK, _zr, 0)

    @pl.when(sid < NS - 1)
    def _():
        for j in range(RPT // CHUNK):
            pltpu.sync_copy(rows0, acc.at[pl.ds(sid * RPT + j * CHUNK, CHUNK)])
            pltpu.sync_copy(rows0.at[0, pl.ds(0, CHUNK)],
                            deg.at[pl.ds(sid * RPT + j * CHUNK, CHUNK)])

    @pl.when(sid == NS - 1)
    def _():
        for j in range((N - (NS - 1) * RPT) // CHUNK):
            pltpu.sync_copy(rows0, acc.at[pl.ds((NS - 1) * RPT + j * CHUNK, CHUNK)])
            pltpu.sync_copy(rows0.at[0, pl.ds(0, CHUNK)],
                            deg.at[pl.ds((NS - 1) * RPT + j * CHUNK, CHUNK)])

    ebase = sid * EPT

    def _stage_col2(s, carry):
        pltpu.sync_copy(coli.at[pl.ds(ebase + s * SUP, SUP)], row_v)

        def _bld(j, carry2):
            for k in range(CHUNK // L):
                col2_v[j, pl.ds(k * L, L)] = row_v[pl.ds(j * CHUNK + k * L, L)]
            return carry2
        lax.fori_loop(0, NCPS, _bld, carry)
        return carry

    plsc.subcore_barrier()

    def _deg_sup(s, carry):
        _stage_col2(s, 0)
        pltpu.sync_copy(wi.at[pl.ds(ebase + s * SUP, SUP)], w_v)

        def _fire(c, carry2):
            pltpu.async_copy(w_v.at[pl.ds(c * CHUNK, CHUNK)],
                             deg.at[col2_v.at[c]], asem, add=True)
            return carry2
        lax.fori_loop(0, NCPS, _fire, 0)

        def _drain(c, carry2):
            pltpu.make_async_copy(w_v.at[pl.ds(c * CHUNK, CHUNK)],
                                  deg.at[col2_v.at[c]], asem).wait()
            return carry2
        lax.fori_loop(0, NCPS, _drain, 0)
        return carry
    lax.fori_loop(0, NSUP, _deg_sup, 0)

    plsc.subcore_barrier()

    pltpu.sync_copy(deg, dis_v)

    def _dis(g, carry):
        d = dis_v[pl.ds(g * L, L)] + 1.0
        i = lax.bitcast_convert_type(d, jnp.int32)
        i = 0x5F3759DF - lax.shift_right_arithmetic(i, 1)
        y = lax.bitcast_convert_type(i, jnp.float32)
        y = y * (1.5 - 0.5 * d * y * y)
        y = y * (1.5 - 0.5 * d * y * y)
        y = y * (1.5 - 0.5 * d * y * y)
        dis_v[pl.ds(g * L, L)] = y
        return carry
    lax.fori_loop(0, N // L, _dis, 0)

    def _gissue(tab, c, p):
        pltpu.async_copy(tab.at[row_v.at[pl.ds(c * CHUNK, CHUNK)]],
                         rows[p], gsems[p])

    def _gwait(tab, c, p):
        pltpu.make_async_copy(tab.at[row_v.at[pl.ds(c * CHUNK, CHUNK)]],
                              rows[p], gsems[p]).wait()

    def _sissue(c, p):
        pltpu.async_copy(rows[p], acc.at[col2_v.at[c]], ssems[p], add=True)

    def _swait(c, p):
        pltpu.make_async_copy(rows[p], acc.at[col2_v.at[c]], ssems[p]).wait()

    def _scale(c, p):
        def _sc1(q, carry):
            e0 = q * 4
            for de in range(4):
                e = e0 + de
                nb = plsc.load_gather(
                    w_v, [jnp.full((L,), c * CHUNK + e, jnp.int32)])
                for j in range(CH // L):
                    rows[p][e, pl.ds(j * L, L)] = rows[p][e, pl.ds(j * L, L)] * nb
            return carry
        lax.fori_loop(0, CHUNK // 4, _sc1, 0)

    def _make_sup(tab):
      def _sup(s, carry):
        _stage_col2(s, 0)
        pltpu.sync_copy(wi.at[pl.ds(ebase + s * SUP, SUP)], w_v)

        def _ncol(j, carry2):
            for k in range(CHUNK // L):
                d0 = pl.ds(j * CHUNK + k * L, L)
                dc = plsc.load_gather(dis_v, [row_v[d0]])
                w_v[d0] = w_v[d0] * dc
            return carry2
        lax.fori_loop(0, NCPS, _ncol, 0)

        pltpu.sync_copy(rowi.at[pl.ds(ebase + s * SUP, SUP)], row_v)

        def _nrow(j, carry2):
            for k in range(CHUNK // L):
                d0 = pl.ds(j * CHUNK + k * L, L)
                dr = plsc.load_gather(dis_v, [row_v[d0]])
                w_v[d0] = w_v[d0] * dr
            return carry2
        lax.fori_loop(0, NCPS, _nrow, 0)

        _gissue(tab, 0, 0)
        _gissue(tab, 1, 1)
        _gwait(tab, 0, 0); _scale(0, 0); _sissue(0, 0)
        _gissue(tab, 2, 2)
        _gwait(tab, 1, 1); _scale(1, 1); _sissue(1, 1)
        _swait(0, 0); _gissue(tab, 3, 0)
        _gwait(tab, 2, 2); _scale(2, 2); _sissue(2, 2)
        _swait(1, 1); _gissue(tab, 4, 1)

        def _tri(t, carry2):
            for k in range(3):
                c = 3 * t + k
                _gwait(tab, c, k)
                _scale(c, k)
                _sissue(c, k)
                _swait(c - 1, (k + 2) % 3)
                _gissue(tab, c + 2, (k + 2) % 3)
            return carry2
        lax.fori_loop(1, NTRI + 1, _tri, 0)

        for c in range(3 * NTRI + 3, NCPS):
            p = c % 3
            _gwait(tab, c, p)
            _scale(c, p)
            _sissue(c, p)
            _swait(c - 1, (c + 2) % 3)
            if c + 2 < NCPS:
                _gissue(tab, c + 2, (c + 2) % 3)
        _swait(NCPS - 1, (NCPS - 1) % 3)
        return carry
      return _sup

    @pl.when(cid == 0)
    def _():
        lax.fori_loop(0, NSUP, _make_sup(xi), 0)

    @pl.when(cid == 1)
    def _():
        lax.fori_loop(0, NSUP, _make_sup(hi), 0)

    plsc.subcore_barrier()

    tail = N - (NS - 1) * RPT

    @pl.when(sid < NS - 1)
    def _():
        pltpu.sync_copy(acc.at[pl.ds(sid * RPT, RPT)],
                        agg_out.at[pl.ds(cid * N + sid * RPT, RPT)])

    @pl.when(sid == NS - 1)
    def _():
        pltpu.sync_copy(acc.at[pl.ds((NS - 1) * RPT, tail)],
                        agg_out.at[pl.ds(cid * N + (NS - 1) * RPT, tail)])

    @pl.when(cid == 0)
    def _():
        @pl.when(sid < NS - 1)
        def _():
            pltpu.sync_copy(dis_v.at[pl.ds(sid * RPT, RPT)],
                            dis_out.at[pl.ds(sid * RPT, RPT)])

        @pl.when(sid == NS - 1)
        def _():
            pltpu.sync_copy(dis_v.at[pl.ds((NS - 1) * RPT, tail)],
                            dis_out.at[pl.ds((NS - 1) * RPT, tail)])


_sc_agg = functools.partial(
    pl.kernel,
    out_type=(
        jax.ShapeDtypeStruct((2 * N, CH), jnp.float32),
        jax.ShapeDtypeStruct((N,), jnp.float32),
    ),
    mesh=plsc.VectorSubcoreMesh(core_axis_name="c", subcore_axis_name="s"),
    compiler_params=pltpu.CompilerParams(needs_layout_passes=False),
    scratch_types=[
        pltpu.VMEM_SHARED((N, CH), jnp.float32),
        pltpu.VMEM_SHARED((N,), jnp.float32),
        pltpu.VMEM((N,), jnp.float32),
        pltpu.VMEM((SUP,), jnp.int32),
        pltpu.VMEM((SUP,), jnp.float32),
        pltpu.VMEM((NCPS, CHUNK), jnp.int32),
        pltpu.VMEM((CHUNK, CH), jnp.float32),
        pltpu.VMEM((CHUNK, CH), jnp.float32),
        pltpu.VMEM((CHUNK, CH), jnp.float32),
        pltpu.SemaphoreType.DMA,
        pltpu.SemaphoreType.DMA,
        pltpu.SemaphoreType.DMA,
        pltpu.SemaphoreType.DMA,
        pltpu.SemaphoreType.DMA,
        pltpu.SemaphoreType.DMA,
        pltpu.SemaphoreType.DMA,
    ],
)(_sc_body)


BN = 2000


def _tc_body(aggx, aggh, x, h, d, ct, wx, wh, bb, hn, cn):
    d2 = d[...] * d[...]
    ax = aggx[...] + d2 * x[...]
    ah = aggh[...] + d2 * h[...]
    cc = jnp.dot(ax, wx[...], preferred_element_type=jnp.float32)
    cc = cc + jnp.dot(ah, wh[...], preferred_element_type=jnp.float32)
    cc = cc + bb[...]
    ig = jax.nn.sigmoid(cc[:, :CH])
    fg = jax.nn.sigmoid(cc[:, CH:2 * CH])
    og = jax.nn.sigmoid(cc[:, 2 * CH:3 * CH])
    gg = jnp.tanh(cc[:, 3 * CH:])
    c2 = fg * ct[...] + ig * gg
    hn[...] = og * jnp.tanh(c2)
    cn[...] = c2


_tc_cell = pl.pallas_call(
    _tc_body,
    grid=(N // BN,),
    in_specs=[
        pl.BlockSpec((BN, CH), lambda i: (i, 0)),
        pl.BlockSpec((BN, CH), lambda i: (N // BN + i, 0)),
        pl.BlockSpec((BN, CH), lambda i: (i, 0)),
        pl.BlockSpec((BN, CH), lambda i: (i, 0)),
        pl.BlockSpec((BN, 1), lambda i: (i, 0)),
        pl.BlockSpec((BN, CH), lambda i: (i, 0)),
        pl.BlockSpec((CH, 4 * CH), lambda i: (0, 0)),
        pl.BlockSpec((CH, 4 * CH), lambda i: (0, 0)),
        pl.BlockSpec((1, 4 * CH), lambda i: (0, 0)),
    ],
    out_specs=[
        pl.BlockSpec((BN, CH), lambda i: (i, 0)),
        pl.BlockSpec((BN, CH), lambda i: (i, 0)),
    ],
    out_shape=[
        jax.ShapeDtypeStruct((N, CH), jnp.float32),
        jax.ShapeDtypeStruct((N, CH), jnp.float32),
    ],
)


def kernel(x, edge_index, edge_attr, h_t, c_t, W, b):
    row = edge_index[0]
    col = edge_index[1]
    agg, dis = _sc_agg(x, h_t, row, col, edge_attr)
    wt = W.T
    hn, cn = _tc_cell(agg, agg, x, h_t, dis[:, None], c_t,
                      wt[:CH], wt[CH:], b[None, :])
    return hn, cn
